# baseline (speedup 1.0000x reference)
import jax
import jax.numpy as jnp
from jax.experimental import pallas as pl
from jax.experimental.pallas import tpu as pltpu

_N = 4096
_H = 128
_BM = 512


def _fused_kernel(pf_hbm, t_hbm, wp_hbm, bp_hbm, wt_hbm, bt_hbm, wo_hbm,
                  bo_hbm, e_ref, out_ref, h_ref, pf_v, t_v, wp_v, bp_v, wt_v,
                  bt_v, wo_v, bo_v, sem):
    m = pl.program_id(0)

    @pl.when(m == 0)
    def _compute_h():
        copies = [
            pltpu.make_async_copy(pf_hbm, pf_v, sem),
            pltpu.make_async_copy(t_hbm, t_v, sem),
            pltpu.make_async_copy(wp_hbm, wp_v, sem),
            pltpu.make_async_copy(bp_hbm, bp_v, sem),
            pltpu.make_async_copy(wt_hbm, wt_v, sem),
            pltpu.make_async_copy(bt_hbm, bt_v, sem),
            pltpu.make_async_copy(wo_hbm, wo_v, sem),
            pltpu.make_async_copy(bo_hbm, bo_v, sem),
        ]
        for c in copies:
            c.start()
        for c in copies:
            c.wait()
        pf_b = pf_v[...].astype(jnp.bfloat16)
        wp_b = wp_v[...].astype(jnp.bfloat16)
        ph = jnp.dot(pf_b, wp_b.T, preferred_element_type=jnp.float32)
        th = t_v[...] * wt_v[...]
        h = ph + bp_v[...] + th + bt_v[...]
        h = jnp.clip(h, -1000000.0, 1000000.0)
        h_ref[...] = h.astype(jnp.bfloat16)
        wo_v[...] = wo_v[...].astype(jnp.bfloat16).astype(jnp.float32)

    e_b = e_ref[...].astype(jnp.bfloat16)
    c = jnp.dot(e_b, h_ref[...], preferred_element_type=jnp.float32)
    c_b = c.astype(jnp.bfloat16).astype(jnp.float32)
    out_ref[...] = jnp.sum(c_b * wo_v[...], axis=1, keepdims=True) + bo_v[...]


def kernel(policy_features, traffic_features, edge_index, W_policy, b_policy,
           W_traffic, b_traffic, W_out, b_out):
    t_col = traffic_features.reshape(_N, 1)
    wt_row = W_traffic.reshape(1, _H)
    bp_row = b_policy.reshape(1, _H)
    bt_row = b_traffic.reshape(1, _H)
    bo_11 = b_out.reshape(1, 1)

    n_blocks = _N // _BM
    hbm_spec = pl.BlockSpec(memory_space=pltpu.MemorySpace.HBM)

    return pl.pallas_call(
        _fused_kernel,
        grid=(n_blocks,),
        in_specs=[
            hbm_spec,
            hbm_spec,
            hbm_spec,
            hbm_spec,
            hbm_spec,
            hbm_spec,
            hbm_spec,
            hbm_spec,
            pl.BlockSpec((_BM, _N), lambda m: (m, 0)),
        ],
        out_specs=pl.BlockSpec((_BM, 1), lambda m: (m, 0)),
        out_shape=jax.ShapeDtypeStruct((_N, 1), jnp.float32),
        scratch_shapes=[
            pltpu.VMEM((_N, _H), jnp.bfloat16),
            pltpu.VMEM((_N, _H), jnp.float32),
            pltpu.VMEM((_N, 1), jnp.float32),
            pltpu.VMEM((_H, _H), jnp.float32),
            pltpu.VMEM((1, _H), jnp.float32),
            pltpu.VMEM((1, _H), jnp.float32),
            pltpu.VMEM((1, _H), jnp.float32),
            pltpu.VMEM((1, _H), jnp.float32),
            pltpu.VMEM((1, 1), jnp.float32),
            pltpu.SemaphoreType.DMA,
        ],
    )(policy_features, t_col, W_policy, bp_row, wt_row, bt_row, W_out, bo_11,
      edge_index)
